# baseline (speedup 1.0000x reference)
import functools

import jax
import jax.numpy as jnp
from jax import lax
from jax.experimental import pallas as pl
from jax.experimental.pallas import tpu as pltpu
from jax.experimental.pallas import tpu_sc as plsc

_BOOST = 3.0
_CB = 16


@functools.lru_cache(maxsize=None)
def _build(NB, S, V, D, nc, ns):
    NW = nc * ns
    assert NB % NW == 0
    nb = NB // NW
    assert nb % (2 * _CB) == 0
    nch = nb // _CB
    lanes = 16
    mesh = plsc.VectorSubcoreMesh(
        core_axis_name="c", subcore_axis_name="s", num_cores=nc, num_subcores=ns
    )

    @functools.partial(
        pl.kernel,
        out_type=jax.ShapeDtypeStruct((NB, S, D), jnp.float32),
        mesh=mesh,
        scratch_types=[
            pltpu.VMEM((nb, S), jnp.int32),
            pltpu.VMEM((_CB, S, D), jnp.float32),
            pltpu.VMEM((_CB, S, D), jnp.float32),
            pltpu.SemaphoreType.DMA,
            pltpu.SemaphoreType.DMA,
        ],
        compiler_params=pltpu.CompilerParams(use_tc_tiling_on_sc=False),
    )
    def k(x_hbm, w_hbm, out_hbm, idx_v, rows0, rows1, sem0, sem1):
        wid = lax.axis_index("s") * nc + lax.axis_index("c")
        base = wid * nb
        pltpu.sync_copy(x_hbm.at[pl.ds(base, nb)], idx_v)

        def gather_start(g, buf, sem):
            for j in range(_CB):
                pltpu.async_copy(
                    w_hbm.at[idx_v.at[g * _CB + j, :]],
                    buf.at[j],
                    sem,
                )

        def gather_wait(buf, sem):
            for j in range(_CB):
                pltpu.make_async_copy(
                    w_hbm.at[idx_v.at[0, :]],
                    buf.at[j],
                    sem,
                ).wait()

        def scale(buf):
            @plsc.parallel_loop(0, _CB * S, 1, unroll=4)
            def _(r):
                i = r // S
                s = r % S
                for c in range(D // lanes):
                    sl = (i, s, pl.ds(c * lanes, lanes))
                    buf[sl] = buf[sl] * _BOOST

        def flush(g, buf):
            pltpu.sync_copy(buf, out_hbm.at[pl.ds(base + g * _CB, _CB)])

        gather_start(0, rows0, sem0)
        gather_start(1, rows1, sem1)

        @pl.loop(0, nch, step=2)
        def _(h):
            gather_wait(rows0, sem0)
            scale(rows0)
            flush(h, rows0)

            @pl.when(h + 2 < nch)
            def _():
                gather_start(h + 2, rows0, sem0)

            gather_wait(rows1, sem1)
            scale(rows1)
            flush(h + 1, rows1)

            @pl.when(h + 3 < nch)
            def _():
                gather_start(h + 3, rows1, sem1)

    return k


def kernel(x, weight):
    V, D = weight.shape
    NB, S = x.shape
    info = plsc.get_sparse_core_info()
    fn = _build(NB, S, V, D, info.num_cores, info.num_subcores)
    return fn(x, weight)
